# Initial kernel scaffold; baseline (speedup 1.0000x reference)
import jax
import jax.numpy as jnp
from jax.experimental import pallas as pl


def kernel(x, state, edge_index, delay, edge_w, funcx_t, t, d, w_param, trans_y_W, trans_y_b, trans_c_W, trans_c_b, gate_W, gate_b, gate_out_W, gate_out_b):
    raise NotImplementedError("write your pallas kernel here")


# speedup vs baseline: 27.3471x; 27.3471x over previous
import functools

import jax
import jax.numpy as jnp
from jax import lax
from jax.experimental import pallas as pl
from jax.experimental.pallas import tpu as pltpu
from jax.experimental.pallas import tpu_sc as plsc

N = 10000
E = 160000
B = 4
HID = 64
FH = (B // 2) * HID
NC = 2
NS = 16
EPT = E // NS
EB = 2000
NBLK = EPT // EB
K = 80
NCHB = EB // K
NP_ = 10240
RPT = NP_ // NS
ZR = 32
TROWS = N * 4


def _sc_segment_sum(table, src, dst, delay, edge_w, tvec):
    mesh = plsc.VectorSubcoreMesh(core_axis_name="c", subcore_axis_name="s")

    @functools.partial(
        pl.kernel,
        out_type=jax.ShapeDtypeStruct((NC, NP_, FH), jnp.float32),
        mesh=mesh,
        compiler_params=pltpu.CompilerParams(needs_layout_passes=False),
        scratch_types=[
            pltpu.VMEM((EB,), jnp.int32),
            pltpu.VMEM((EB,), jnp.int32),
            pltpu.VMEM((EB,), jnp.float32),
            pltpu.VMEM((EB,), jnp.float32),
            pltpu.VMEM((NCHB, K), jnp.int32),
            pltpu.VMEM((NCHB, K), jnp.int32),
            pltpu.VMEM((K, FH), jnp.float32),
            pltpu.VMEM((16,), jnp.int32),
            pltpu.VMEM((ZR, FH), jnp.float32),
            pltpu.VMEM_SHARED((NP_, FH), jnp.float32),
            pltpu.SemaphoreType.DMA,
        ],
    )
    def k(table_h, src_h, dst_h, dl_h, ew_h, tv_h, out_h,
          src_v, dst_v, dl_v, ew_v, gidx_v, didx_v, rows_v, tv_v, zbuf,
          acc, sem):
        c = lax.axis_index("c")
        s = lax.axis_index("s")
        eb = s * EPT
        pltpu.sync_copy(tv_h, tv_v)
        t16 = tv_v[...]
        tf16 = t16.astype(jnp.float32)
        ti16 = t16 * 4
        half_off = c * TROWS

        zv = jnp.zeros((16,), jnp.float32)

        def z_body(r, carry):
            for j in range(FH // 16):
                zbuf[r, pl.ds(j * 16, 16)] = zv
            return carry

        lax.fori_loop(0, ZR, z_body, 0)
        rb = s * RPT
        for kk in range(RPT // ZR):
            pltpu.sync_copy(zbuf, acc.at[pl.ds(rb + kk * ZR, ZR)])
        plsc.subcore_barrier()

        def blk_body(blk, carry):
            ebb = eb + blk * EB
            pltpu.sync_copy(src_h.at[pl.ds(ebb, EB)], src_v)
            pltpu.sync_copy(dst_h.at[pl.ds(ebb, EB)], dst_v)
            pltpu.sync_copy(dl_h.at[pl.ds(ebb, EB)], dl_v)
            pltpu.sync_copy(ew_h.at[pl.ds(ebb, EB)], ew_v)

            def idx_body(ch, carry1):
                for j in range(K // 16):
                    sl = pl.ds(ch * K + j * 16, 16)
                    sv = src_v[sl]
                    dv = dl_v[sl]
                    catch = ((tf16 + dv) * 4.0).astype(jnp.int32)
                    cidx = jnp.clip(catch - ti16, 0, 3)
                    gidx_v[ch, pl.ds(j * 16, 16)] = sv * 4 + cidx + half_off
                    didx_v[ch, pl.ds(j * 16, 16)] = dst_v[sl]
                return carry1

            lax.fori_loop(0, NCHB, idx_body, 0)

            def chunk_body(ch, carry1):
                pltpu.async_copy(table_h.at[gidx_v.at[ch]], rows_v,
                                 sem).wait()

                def scale_body(e, carry2):
                    wi = jnp.full((16,), ch * K + e, jnp.int32)
                    w16 = plsc.load_gather(ew_v, [wi])
                    for j in range(FH // 16):
                        sl = pl.ds(j * 16, 16)
                        rows_v[e, sl] = rows_v[e, sl] * w16
                    return carry2

                lax.fori_loop(0, K, scale_body, 0)
                pltpu.sync_copy(rows_v, acc.at[didx_v.at[ch]], add=True)
                return carry1

            lax.fori_loop(0, NCHB, chunk_body, 0)
            return carry

        lax.fori_loop(0, NBLK, blk_body, 0)
        plsc.subcore_barrier()
        pltpu.sync_copy(acc.at[pl.ds(rb, RPT)], out_h.at[c, pl.ds(rb, RPT)])

    return k(table, src, dst, delay, edge_w, tvec)


def _tc_tail(s2, x2, dx2, d, w_param, ty_Wt, ty_b, tc_Wt, tc_b,
             g_Wt, g_b, go_Wt, go_b):
    R = N * B
    BR = 4000

    def body(s_ref, x_ref, dx_ref, d_ref, wp_ref, tyW_ref, tyb_ref,
             tcW_ref, tcb_ref, gW_ref, gb_ref, goW_ref, gob_ref, o_ref):
        d_c = jnp.clip(d_ref[...], 0.0, 1.0)
        wp = wp_ref[...]
        w = jnp.dot(wp * d_c, wp.T, preferred_element_type=jnp.float32)
        y = jnp.dot(s_ref[...], w, preferred_element_type=jnp.float32)
        gate = jax.nn.sigmoid(
            jnp.dot(y, gW_ref[...], preferred_element_type=jnp.float32)
            + gb_ref[...])
        y = (1.0 - gate) * (y - x_ref[...])
        dxc = jnp.dot(dx_ref[...], tcW_ref[...],
                      preferred_element_type=jnp.float32) + tcb_ref[...]
        y = y * dxc
        y2 = jnp.dot(jnp.maximum(y, 0.0), tyW_ref[...],
                     preferred_element_type=jnp.float32) + tyb_ref[...]
        o_ref[...] = jax.nn.sigmoid(
            jnp.dot(y2, goW_ref[...], preferred_element_type=jnp.float32)
            + gob_ref[...]) * y2

    row_spec = pl.BlockSpec((BR, HID), lambda i: (i, 0))
    vec_spec = pl.BlockSpec((1, HID), lambda i: (0, 0))
    mat_spec = pl.BlockSpec((HID, HID), lambda i: (0, 0))
    out = pl.pallas_call(
        body,
        grid=(R // BR,),
        in_specs=[row_spec, row_spec, row_spec, vec_spec, mat_spec,
                  mat_spec, vec_spec, mat_spec, vec_spec,
                  mat_spec, vec_spec, mat_spec, vec_spec],
        out_specs=row_spec,
        out_shape=jax.ShapeDtypeStruct((R, HID), jnp.float32),
    )(s2, x2, dx2, d, w_param, ty_Wt, ty_b, tc_Wt, tc_b,
      g_Wt, g_b, go_Wt, go_b)
    return out


def kernel(x, state, edge_index, delay, edge_w, funcx_t, t, d, w_param,
           trans_y_W, trans_y_b, trans_c_W, trans_c_b, gate_W, gate_b,
           gate_out_W, gate_out_b):
    t_i = jnp.asarray(t, jnp.int32)
    ti = t_i * 4
    tail = lax.dynamic_slice_in_dim(state, ti + 1, 3, axis=1)
    table = jnp.concatenate([x[:, None], tail], axis=1)
    table = table.reshape(TROWS, NC, FH).transpose(1, 0, 2).reshape(
        NC * TROWS, FH)
    src = edge_index[0].astype(jnp.int32)
    dst = edge_index[1].astype(jnp.int32)
    tvec = jnp.full((16,), t_i, jnp.int32)
    sh = _sc_segment_sum(table, src, dst, delay.astype(jnp.float32),
                         edge_w.astype(jnp.float32), tvec)
    s2 = sh[:, :N].reshape(NC, N, NC, HID).transpose(1, 0, 2, 3).reshape(
        N * B, HID)
    x2 = x.reshape(N * B, HID)
    dx2 = jnp.transpose(funcx_t, (1, 0, 2)).reshape(N * B, HID)
    out = _tc_tail(
        s2, x2, dx2,
        d.reshape(1, HID), w_param,
        trans_y_W.T, trans_y_b.reshape(1, HID),
        trans_c_W.T, trans_c_b.reshape(1, HID),
        gate_W.T, gate_b.reshape(1, HID),
        gate_out_W.T, gate_out_b.reshape(1, HID))
    return out.reshape(N, B, HID)


# speedup vs baseline: 42.2764x; 1.5459x over previous
import functools

import jax
import jax.numpy as jnp
from jax import lax
from jax.experimental import pallas as pl
from jax.experimental.pallas import tpu as pltpu
from jax.experimental.pallas import tpu_sc as plsc

N = 10000
E = 160000
B = 4
HID = 64
FH = (B // 2) * HID
NC = 2
NS = 16
EPT = E // NS
EB = 2000
NBLK = EPT // EB
K = 80
NCHB = EB // K
NP_ = 10240
RPT = NP_ // NS
ZR = 32
TROWS = N * 4


def _sc_segment_sum(table, src, dst, delay, edge_w, tvec):
    mesh = plsc.VectorSubcoreMesh(core_axis_name="c", subcore_axis_name="s")

    @functools.partial(
        pl.kernel,
        out_type=jax.ShapeDtypeStruct((NP_, NC, FH), jnp.float32),
        mesh=mesh,
        compiler_params=pltpu.CompilerParams(needs_layout_passes=False),
        scratch_types=[
            pltpu.VMEM((EB,), jnp.int32),
            pltpu.VMEM((EB,), jnp.int32),
            pltpu.VMEM((EB,), jnp.float32),
            pltpu.VMEM((EB,), jnp.float32),
            pltpu.VMEM((NCHB, K), jnp.int32),
            pltpu.VMEM((NCHB, K), jnp.int32),
            pltpu.VMEM((K, FH), jnp.float32),
            pltpu.VMEM((K, FH), jnp.float32),
            pltpu.VMEM((16,), jnp.int32),
            pltpu.VMEM((ZR, FH), jnp.float32),
            pltpu.VMEM_SHARED((NP_, FH), jnp.float32),
            pltpu.SemaphoreType.DMA,
            pltpu.SemaphoreType.DMA,
        ],
    )
    def k(table_h, src_h, dst_h, dl_h, ew_h, tv_h, out_h,
          src_v, dst_v, dl_v, ew_v, gidx_v, didx_v, rows0_v, rows1_v, tv_v,
          zbuf, acc, sem0, sem1):
        c = lax.axis_index("c")
        s = lax.axis_index("s")
        eb = s * EPT
        pltpu.sync_copy(tv_h, tv_v)
        t16 = tv_v[...]
        tf16 = t16.astype(jnp.float32)
        ti16 = t16 * 4

        zv = jnp.zeros((16,), jnp.float32)

        def z_body(r, carry):
            for j in range(FH // 16):
                zbuf[r, pl.ds(j * 16, 16)] = zv
            return carry

        lax.fori_loop(0, ZR, z_body, 0)
        rb = s * RPT
        for kk in range(RPT // ZR):
            pltpu.sync_copy(zbuf, acc.at[pl.ds(rb + kk * ZR, ZR)])
        plsc.subcore_barrier()

        def blk_body(blk, carry):
            ebb = eb + blk * EB
            pltpu.sync_copy(src_h.at[pl.ds(ebb, EB)], src_v)
            pltpu.sync_copy(dst_h.at[pl.ds(ebb, EB)], dst_v)
            pltpu.sync_copy(dl_h.at[pl.ds(ebb, EB)], dl_v)
            pltpu.sync_copy(ew_h.at[pl.ds(ebb, EB)], ew_v)

            def idx_body(ch, carry1):
                for j in range(K // 16):
                    sl = pl.ds(ch * K + j * 16, 16)
                    sv = src_v[sl]
                    dv = dl_v[sl]
                    catch = ((tf16 + dv) * 4.0).astype(jnp.int32)
                    cidx = jnp.clip(catch - ti16, 0, 3)
                    gidx_v[ch, pl.ds(j * 16, 16)] = sv * 8 + cidx * 2 + c
                    didx_v[ch, pl.ds(j * 16, 16)] = dst_v[sl]
                return carry1

            lax.fori_loop(0, NCHB, idx_body, 0)

            def make_scale(rows_v, ch):
                def grp_body(g, carry2):
                    base = g * 16
                    wgrp = ew_v[pl.ds(ch * K + base, 16)]
                    for i in range(16):
                        w16 = wgrp.at[jnp.full((16,), i, jnp.int32)].get(
                            mode='promise_in_bounds')
                        for j in range(FH // 16):
                            sl = pl.ds(j * 16, 16)
                            rows_v[base + i, sl] = rows_v[base + i, sl] * w16
                    return carry2
                return grp_body

            def process(ch, rows_v, other_rows_v, sem, other_sem):
                @pl.when(ch + 1 < NCHB)
                def _():
                    pltpu.async_copy(table_h.at[gidx_v.at[ch + 1]],
                                     other_rows_v, other_sem)

                pltpu.make_async_copy(table_h.at[gidx_v.at[ch]], rows_v,
                                      sem).wait()
                lax.fori_loop(0, K // 16, make_scale(rows_v, ch), 0)
                pltpu.sync_copy(rows_v, acc.at[didx_v.at[ch]], add=True)

            pltpu.async_copy(table_h.at[gidx_v.at[0]], rows0_v, sem0)

            def chunk_body(ch, carry1):
                even = lax.rem(ch, 2) == 0

                @pl.when(even)
                def _():
                    process(ch, rows0_v, rows1_v, sem0, sem1)

                @pl.when(jnp.logical_not(even))
                def _():
                    process(ch, rows1_v, rows0_v, sem1, sem0)

                return carry1

            lax.fori_loop(0, NCHB, chunk_body, 0)
            return carry

        lax.fori_loop(0, NBLK, blk_body, 0)
        plsc.subcore_barrier()
        pltpu.sync_copy(acc.at[pl.ds(rb, RPT)], out_h.at[pl.ds(rb, RPT), c])

    return k(table, src, dst, delay, edge_w, tvec)


def _tc_tail(s2, x2, dx2, d, w_param, ty_Wt, ty_b, tc_Wt, tc_b,
             g_Wt, g_b, go_Wt, go_b):
    R = N * B
    BR = 4000

    def body(s_ref, x_ref, dx_ref, d_ref, wp_ref, tyW_ref, tyb_ref,
             tcW_ref, tcb_ref, gW_ref, gb_ref, goW_ref, gob_ref, o_ref):
        d_c = jnp.clip(d_ref[...], 0.0, 1.0)
        wp = wp_ref[...]
        w = jnp.dot(wp * d_c, wp.T, preferred_element_type=jnp.float32)
        y = jnp.dot(s_ref[...], w, preferred_element_type=jnp.float32)
        gate = jax.nn.sigmoid(
            jnp.dot(y, gW_ref[...], preferred_element_type=jnp.float32)
            + gb_ref[...])
        y = (1.0 - gate) * (y - x_ref[...])
        dxc = jnp.dot(dx_ref[...], tcW_ref[...],
                      preferred_element_type=jnp.float32) + tcb_ref[...]
        y = y * dxc
        y2 = jnp.dot(jnp.maximum(y, 0.0), tyW_ref[...],
                     preferred_element_type=jnp.float32) + tyb_ref[...]
        o_ref[...] = jax.nn.sigmoid(
            jnp.dot(y2, goW_ref[...], preferred_element_type=jnp.float32)
            + gob_ref[...]) * y2

    row_spec = pl.BlockSpec((BR, HID), lambda i: (i, 0))
    vec_spec = pl.BlockSpec((1, HID), lambda i: (0, 0))
    mat_spec = pl.BlockSpec((HID, HID), lambda i: (0, 0))
    out = pl.pallas_call(
        body,
        grid=(R // BR,),
        in_specs=[row_spec, row_spec, row_spec, vec_spec, mat_spec,
                  mat_spec, vec_spec, mat_spec, vec_spec,
                  mat_spec, vec_spec, mat_spec, vec_spec],
        out_specs=row_spec,
        out_shape=jax.ShapeDtypeStruct((R, HID), jnp.float32),
    )(s2, x2, dx2, d, w_param, ty_Wt, ty_b, tc_Wt, tc_b,
      g_Wt, g_b, go_Wt, go_b)
    return out


def kernel(x, state, edge_index, delay, edge_w, funcx_t, t, d, w_param,
           trans_y_W, trans_y_b, trans_c_W, trans_c_b, gate_W, gate_b,
           gate_out_W, gate_out_b):
    t_i = jnp.asarray(t, jnp.int32)
    ti = t_i * 4
    tail = lax.dynamic_slice_in_dim(state, ti + 1, 3, axis=1)
    table = jnp.concatenate([x[:, None], tail], axis=1)
    table = table.reshape(NC * TROWS, FH)
    src = edge_index[0].astype(jnp.int32)
    dst = edge_index[1].astype(jnp.int32)
    tvec = jnp.full((16,), t_i, jnp.int32)
    sh = _sc_segment_sum(table, src, dst, delay.astype(jnp.float32),
                         edge_w.astype(jnp.float32), tvec)
    s2 = sh[:N].reshape(N * B, HID)
    x2 = x.reshape(N * B, HID)
    dx2 = jnp.transpose(funcx_t, (1, 0, 2)).reshape(N * B, HID)
    out = _tc_tail(
        s2, x2, dx2,
        d.reshape(1, HID), w_param,
        trans_y_W.T, trans_y_b.reshape(1, HID),
        trans_c_W.T, trans_c_b.reshape(1, HID),
        gate_W.T, gate_b.reshape(1, HID),
        gate_out_W.T, gate_out_b.reshape(1, HID))
    return out.reshape(N, B, HID)


# speedup vs baseline: 42.8884x; 1.0145x over previous
import functools

import jax
import jax.numpy as jnp
from jax import lax
from jax.experimental import pallas as pl
from jax.experimental.pallas import tpu as pltpu
from jax.experimental.pallas import tpu_sc as plsc

N = 10000
E = 160000
B = 4
HID = 64
FH = (B // 2) * HID
NC = 2
NS = 16
EPT = E // NS
EB = 2000
NBLK = EPT // EB
K = 80
NCHB = EB // K
NP_ = 10240
RPT = NP_ // NS
ZR = 32
TROWS = N * 4


def _sc_segment_sum(table, src, dst, delay, edge_w, tvec):
    mesh = plsc.VectorSubcoreMesh(core_axis_name="c", subcore_axis_name="s")

    @functools.partial(
        pl.kernel,
        out_type=jax.ShapeDtypeStruct((NP_, NC, FH), jnp.float32),
        mesh=mesh,
        compiler_params=pltpu.CompilerParams(needs_layout_passes=False),
        scratch_types=[
            pltpu.VMEM((EB,), jnp.int32),
            pltpu.VMEM((EB,), jnp.int32),
            pltpu.VMEM((EB,), jnp.float32),
            pltpu.VMEM((EB,), jnp.float32),
            pltpu.VMEM((NCHB, K), jnp.int32),
            pltpu.VMEM((NCHB, K), jnp.int32),
            pltpu.VMEM((K, FH), jnp.float32),
            pltpu.VMEM((K, FH), jnp.float32),
            pltpu.VMEM((16,), jnp.int32),
            pltpu.VMEM((ZR, FH), jnp.float32),
            pltpu.VMEM_SHARED((NP_, FH), jnp.float32),
            pltpu.SemaphoreType.DMA,
            pltpu.SemaphoreType.DMA,
            pltpu.SemaphoreType.DMA,
            pltpu.SemaphoreType.DMA,
        ],
    )
    def k(table_h, src_h, dst_h, dl_h, ew_h, tv_h, out_h,
          src_v, dst_v, dl_v, ew_v, gidx_v, didx_v, rows0_v, rows1_v, tv_v,
          zbuf, acc, sem0, sem1, ssem0, ssem1):
        c = lax.axis_index("c")
        s = lax.axis_index("s")
        eb = s * EPT
        pltpu.sync_copy(tv_h, tv_v)
        t16 = tv_v[...]
        tf16 = t16.astype(jnp.float32)
        ti16 = t16 * 4

        zv = jnp.zeros((16,), jnp.float32)

        def z_body(r, carry):
            for j in range(FH // 16):
                zbuf[r, pl.ds(j * 16, 16)] = zv
            return carry

        lax.fori_loop(0, ZR, z_body, 0)
        rb = s * RPT
        for kk in range(RPT // ZR):
            pltpu.sync_copy(zbuf, acc.at[pl.ds(rb + kk * ZR, ZR)])
        plsc.subcore_barrier()

        def blk_body(blk, carry):
            @pl.when(blk > 0)
            def _():
                pltpu.make_async_copy(
                    rows0_v, acc.at[didx_v.at[NCHB - 1]], ssem0).wait()

            ebb = eb + blk * EB
            pltpu.sync_copy(src_h.at[pl.ds(ebb, EB)], src_v)
            pltpu.sync_copy(dst_h.at[pl.ds(ebb, EB)], dst_v)
            pltpu.sync_copy(dl_h.at[pl.ds(ebb, EB)], dl_v)
            pltpu.sync_copy(ew_h.at[pl.ds(ebb, EB)], ew_v)

            def idx_body(ch, carry1):
                for j in range(K // 16):
                    sl = pl.ds(ch * K + j * 16, 16)
                    sv = src_v[sl]
                    dv = dl_v[sl]
                    catch = ((tf16 + dv) * 4.0).astype(jnp.int32)
                    cidx = jnp.clip(catch - ti16, 0, 3)
                    gidx_v[ch, pl.ds(j * 16, 16)] = sv * 8 + cidx * 2 + c
                    didx_v[ch, pl.ds(j * 16, 16)] = dst_v[sl]
                return carry1

            lax.fori_loop(0, NCHB, idx_body, 0)

            def make_scale(rows_v, ch):
                def grp_body(g, carry2):
                    base = g * 16
                    wgrp = ew_v[pl.ds(ch * K + base, 16)]
                    for i in range(16):
                        w16 = wgrp.at[jnp.full((16,), i, jnp.int32)].get(
                            mode='promise_in_bounds')
                        for j in range(FH // 16):
                            sl = pl.ds(j * 16, 16)
                            rows_v[base + i, sl] = rows_v[base + i, sl] * w16
                    return carry2
                return grp_body

            def process(ch, rows_v, other_rows_v, sem, other_sem,
                        ssem, other_ssem):
                @pl.when(ch >= 1)
                def _():
                    pltpu.make_async_copy(
                        other_rows_v, acc.at[didx_v.at[ch - 1]],
                        other_ssem).wait()

                @pl.when(ch + 1 < NCHB)
                def _():
                    pltpu.async_copy(table_h.at[gidx_v.at[ch + 1]],
                                     other_rows_v, other_sem)

                pltpu.make_async_copy(table_h.at[gidx_v.at[ch]], rows_v,
                                      sem).wait()
                lax.fori_loop(0, K // 16, make_scale(rows_v, ch), 0)
                pltpu.async_copy(rows_v, acc.at[didx_v.at[ch]], ssem,
                                 add=True)

            pltpu.async_copy(table_h.at[gidx_v.at[0]], rows0_v, sem0)

            def chunk_body(ch, carry1):
                even = lax.rem(ch, 2) == 0

                @pl.when(even)
                def _():
                    process(ch, rows0_v, rows1_v, sem0, sem1, ssem0, ssem1)

                @pl.when(jnp.logical_not(even))
                def _():
                    process(ch, rows1_v, rows0_v, sem1, sem0, ssem1, ssem0)

                return carry1

            lax.fori_loop(0, NCHB, chunk_body, 0)
            return carry

        lax.fori_loop(0, NBLK, blk_body, 0)
        pltpu.make_async_copy(rows0_v, acc.at[didx_v.at[NCHB - 1]],
                              ssem0).wait()
        plsc.subcore_barrier()
        pltpu.sync_copy(acc.at[pl.ds(rb, RPT)], out_h.at[pl.ds(rb, RPT), c])

    return k(table, src, dst, delay, edge_w, tvec)


def _tc_tail(s2, x2, dx2, d, w_param, ty_Wt, ty_b, tc_Wt, tc_b,
             g_Wt, g_b, go_Wt, go_b):
    R = N * B
    BR = 4000

    def body(s_ref, x_ref, dx_ref, d_ref, wp_ref, tyW_ref, tyb_ref,
             tcW_ref, tcb_ref, gW_ref, gb_ref, goW_ref, gob_ref, o_ref):
        d_c = jnp.clip(d_ref[...], 0.0, 1.0)
        wp = wp_ref[...]
        w = jnp.dot(wp * d_c, wp.T, preferred_element_type=jnp.float32)
        y = jnp.dot(s_ref[...], w, preferred_element_type=jnp.float32)
        gate = jax.nn.sigmoid(
            jnp.dot(y, gW_ref[...], preferred_element_type=jnp.float32)
            + gb_ref[...])
        y = (1.0 - gate) * (y - x_ref[...])
        dxc = jnp.dot(dx_ref[...], tcW_ref[...],
                      preferred_element_type=jnp.float32) + tcb_ref[...]
        y = y * dxc
        y2 = jnp.dot(jnp.maximum(y, 0.0), tyW_ref[...],
                     preferred_element_type=jnp.float32) + tyb_ref[...]
        o_ref[...] = jax.nn.sigmoid(
            jnp.dot(y2, goW_ref[...], preferred_element_type=jnp.float32)
            + gob_ref[...]) * y2

    row_spec = pl.BlockSpec((BR, HID), lambda i: (i, 0))
    vec_spec = pl.BlockSpec((1, HID), lambda i: (0, 0))
    mat_spec = pl.BlockSpec((HID, HID), lambda i: (0, 0))
    out = pl.pallas_call(
        body,
        grid=(R // BR,),
        in_specs=[row_spec, row_spec, row_spec, vec_spec, mat_spec,
                  mat_spec, vec_spec, mat_spec, vec_spec,
                  mat_spec, vec_spec, mat_spec, vec_spec],
        out_specs=row_spec,
        out_shape=jax.ShapeDtypeStruct((R, HID), jnp.float32),
    )(s2, x2, dx2, d, w_param, ty_Wt, ty_b, tc_Wt, tc_b,
      g_Wt, g_b, go_Wt, go_b)
    return out


def kernel(x, state, edge_index, delay, edge_w, funcx_t, t, d, w_param,
           trans_y_W, trans_y_b, trans_c_W, trans_c_b, gate_W, gate_b,
           gate_out_W, gate_out_b):
    t_i = jnp.asarray(t, jnp.int32)
    ti = t_i * 4
    tail = lax.dynamic_slice_in_dim(state, ti + 1, 3, axis=1)
    table = jnp.concatenate([x[:, None], tail], axis=1)
    table = table.reshape(NC * TROWS, FH)
    src = edge_index[0].astype(jnp.int32)
    dst = edge_index[1].astype(jnp.int32)
    tvec = jnp.full((16,), t_i, jnp.int32)
    sh = _sc_segment_sum(table, src, dst, delay.astype(jnp.float32),
                         edge_w.astype(jnp.float32), tvec)
    s2 = sh.reshape(NP_ * B, HID)
    x2 = x.reshape(N * B, HID)
    dx2 = jnp.transpose(funcx_t, (1, 0, 2)).reshape(N * B, HID)
    out = _tc_tail(
        s2, x2, dx2,
        d.reshape(1, HID), w_param,
        trans_y_W.T, trans_y_b.reshape(1, HID),
        trans_c_W.T, trans_c_b.reshape(1, HID),
        gate_W.T, gate_b.reshape(1, HID),
        gate_out_W.T, gate_out_b.reshape(1, HID))
    return out.reshape(N, B, HID)
